# baseline (speedup 1.0000x reference)
import numpy as np
import jax
import jax.numpy as jnp
from jax.experimental import pallas as pl
from jax.experimental.pallas import tpu as pltpu

NB = 512
MINV, MAXV = -1.0, 1.0
BW = (MAXV - MINV) / (NB - 1)
INV_BW = 1.0 / BW
EPSV = 1e-8
G = 8
PS = 32
NPIX = PS * PS
NPAT = 256
NUM_TILES = 85
BATCH = 4
GROUP_OFF = (0, 8, 24, 88)
GROUP_N = (4, 16, 64, 256)
ROWS_PAD = 344


def _pool_matrix() -> np.ndarray:
    a = np.zeros((ROWS_PAD, NPAT), np.float32)
    for s in range(4):
        g = 2 ** s
        f = G // g
        r = GROUP_OFF[s]
        for b in range(BATCH):
            for ci in range(g):
                for cj in range(g):
                    for gi in range(ci * f, (ci + 1) * f):
                        for gj in range(cj * f, (cj + 1) * f):
                            a[r, b * (G * G) + gi * G + gj] = 1.0
                    r += 1
        assert r == GROUP_OFF[s] + GROUP_N[s]
    return a


_A_POOL = _pool_matrix()


def _hist_body(x_ref, out_ref):
    j = pl.program_id(0)
    k = pl.program_id(1)
    cs0 = jax.lax.broadcasted_iota(
        jnp.int32, (8, 128), 0).astype(jnp.float32)
    UN = 128
    NT = 17

    def body(i, accs):
        xs = ((x_ref[pl.ds(i * UN, UN), :] - jnp.float32(MINV - BW))
              * jnp.float32(INV_BW) - (k * 128).astype(jnp.float32))
        out = list(accs)
        for r in range(UN):
            t = jnp.broadcast_to(xs[r:r + 1, :], (8, 128)) - cs0
            for v in range(NT):
                out[v] = out[v] + jax.lax.clamp(
                    jnp.float32(-0.5),
                    t - jnp.float32(8 * v + 0.5),
                    jnp.float32(0.5))
        return tuple(out)

    accs = jax.lax.fori_loop(
        0, NPIX // UN, body, (jnp.zeros((8, 128), jnp.float32),) * NT)
    for v in range(16):
        shifted = jnp.concatenate(
            [accs[v][1:8, :], accs[v + 1][0:1, :]], axis=0)
        out_ref[pl.ds(k * 128 + v * 8, 8), pl.ds(j * 128, 128)] = (
            accs[v] - shifted)


def _fused_body(t0_ref, t1_ref, t2_ref, t3_ref, x_ref, a_ref,
                o_ref, h_ref, tp_ref):
    _hist_body(x_ref, h_ref)
    j = pl.program_id(0)
    k = pl.program_id(1)

    @pl.when(jnp.logical_and(j == pl.num_programs(0) - 1,
                             k == pl.num_programs(1) - 1))
    def _loss_phase():
        tp_ref[...] = jnp.zeros((ROWS_PAD, NB), jnp.float32)
        for off, t_ref in zip(GROUP_OFF, (t0_ref, t1_ref, t2_ref, t3_ref)):
            tg = t_ref[...]
            tp_ref[off:off + tg.shape[0], :] = tg * (
                1.0 / (jnp.sum(tg, axis=1, keepdims=True) + EPSV))
        hall = jax.lax.dot_general(
            a_ref[...], h_ref[...], (((1,), (1,)), ((), ())),
            preferred_element_type=jnp.float32)
        gs = jnp.sum(hall, axis=1, keepdims=True)
        d = hall * (1.0 / (gs + EPSV)) - tp_ref[...]
        ii = jax.lax.broadcasted_iota(jnp.int32, (NB, NB), 0)
        jj = jax.lax.broadcasted_iota(jnp.int32, (NB, NB), 1)
        upper = jnp.where(ii <= jj, 1.0, 0.0)
        cd = jnp.dot(d, upper,
                     preferred_element_type=jnp.float32)
        tot = (jnp.sum(jnp.abs(cd), keepdims=True)
               + jnp.sum(jnp.abs(d), keepdims=True))
        o_ref[...] = tot * jnp.float32(1.0 / (NB * BATCH * NUM_TILES))


def kernel(generated, tgt_s0, tgt_s1, tgt_s2, tgt_s3):
    b = generated.shape[0]
    x = (generated.reshape(b, G, PS, G, PS)
         .transpose(0, 1, 3, 2, 4)
         .reshape(b * G * G, NPIX)
         .T)

    tgts = (tgt_s0.reshape(b, NB), tgt_s1.reshape(b * 4, NB),
            tgt_s2.reshape(b * 16, NB), tgt_s3.reshape(b * 64, NB))

    out = pl.pallas_call(
        _fused_body,
        grid=(NPAT // 128, 4),
        in_specs=[
            *(pl.BlockSpec((t.shape[0], NB), lambda j, k: (0, 0))
              for t in tgts),
            pl.BlockSpec((NPIX, 128), lambda j, k: (0, j)),
            pl.BlockSpec((ROWS_PAD, NPAT), lambda j, k: (0, 0)),
        ],
        out_specs=pl.BlockSpec((1, 1), lambda j, k: (0, 0)),
        out_shape=jax.ShapeDtypeStruct((1, 1), jnp.float32),
        scratch_shapes=[pltpu.VMEM((NB, NPAT), jnp.float32),
                        pltpu.VMEM((ROWS_PAD, NB), jnp.float32)],
        compiler_params=pltpu.CompilerParams(
            dimension_semantics=("arbitrary", "arbitrary")),
    )(*tgts, x, jnp.asarray(_A_POOL))
    return out[0, 0]


# speedup vs baseline: 1.0004x; 1.0004x over previous
import numpy as np
import jax
import jax.numpy as jnp
from jax.experimental import pallas as pl
from jax.experimental.pallas import tpu as pltpu

NB = 512
MINV, MAXV = -1.0, 1.0
BW = (MAXV - MINV) / (NB - 1)
INV_BW = 1.0 / BW
EPSV = 1e-8
G = 8
PS = 32
NPIX = PS * PS
NPAT = 256
NUM_TILES = 85
BATCH = 4
GROUP_OFF = (0, 8, 24, 88)
GROUP_N = (4, 16, 64, 256)
ROWS_PAD = 344


def _pool_matrix() -> np.ndarray:
    a = np.zeros((ROWS_PAD, NPAT), np.float32)
    for s in range(4):
        g = 2 ** s
        f = G // g
        r = GROUP_OFF[s]
        for b in range(BATCH):
            for ci in range(g):
                for cj in range(g):
                    for gi in range(ci * f, (ci + 1) * f):
                        for gj in range(cj * f, (cj + 1) * f):
                            a[r, b * (G * G) + gi * G + gj] = 1.0
                    r += 1
        assert r == GROUP_OFF[s] + GROUP_N[s]
    return a


_A_POOL = _pool_matrix()


def _hist_body(x_ref, out_ref):
    j = pl.program_id(0)
    k = pl.program_id(1)
    cs0 = jax.lax.broadcasted_iota(
        jnp.int32, (8, 128), 0).astype(jnp.float32)
    UN = 128
    NT = 17

    def body(i, accs):
        xs = ((x_ref[pl.ds(i * UN, UN), :] - jnp.float32(MINV - BW))
              * jnp.float32(INV_BW) - (k * 128).astype(jnp.float32))
        out = list(accs)
        for r in range(UN):
            t = jnp.broadcast_to(xs[r:r + 1, :], (8, 128)) - cs0
            for v in range(NT):
                out[v] = out[v] + jax.lax.clamp(
                    jnp.float32(-0.5),
                    t - jnp.float32(8 * v + 0.5),
                    jnp.float32(0.5))
        return tuple(out)

    accs = jax.lax.fori_loop(
        0, NPIX // UN, body, (jnp.zeros((8, 128), jnp.float32),) * NT)
    for v in range(16):
        shifted = jnp.concatenate(
            [accs[v][1:8, :], accs[v + 1][0:1, :]], axis=0)
        out_ref[pl.ds(k * 128 + v * 8, 8), pl.ds(j * 128, 128)] = (
            accs[v] - shifted)


def _fused_body(t0_ref, t1_ref, t2_ref, t3_ref, x_ref, a_ref,
                o_ref, h_ref, tp_ref):
    _hist_body(x_ref, h_ref)
    j = pl.program_id(0)
    k = pl.program_id(1)

    @pl.when(jnp.logical_and(j == pl.num_programs(0) - 1,
                             k == pl.num_programs(1) - 1))
    def _loss_phase():
        tp_ref[...] = jnp.zeros((ROWS_PAD, NB), jnp.float32)
        for off, t_ref in zip(GROUP_OFF, (t0_ref, t1_ref, t2_ref, t3_ref)):
            tg = t_ref[...]
            tp_ref[off:off + tg.shape[0], :] = tg * (
                1.0 / (jnp.sum(tg, axis=1, keepdims=True) + EPSV))
        hall = jax.lax.dot_general(
            a_ref[...], h_ref[...], (((1,), (1,)), ((), ())),
            preferred_element_type=jnp.float32)
        gs = jnp.sum(hall, axis=1, keepdims=True)
        d = hall * (1.0 / (gs + EPSV)) - tp_ref[...]
        ii = jax.lax.broadcasted_iota(jnp.int32, (NB, NB), 0)
        jj = jax.lax.broadcasted_iota(jnp.int32, (NB, NB), 1)
        upper = jnp.where(ii <= jj, 1.0, 0.0)
        cd = jnp.dot(d, upper,
                     preferred_element_type=jnp.float32)
        tot = (jnp.sum(jnp.abs(cd), keepdims=True)
               + jnp.sum(jnp.abs(d), keepdims=True))
        o_ref[...] = tot * jnp.float32(1.0 / (NB * BATCH * NUM_TILES))


def kernel(generated, tgt_s0, tgt_s1, tgt_s2, tgt_s3):
    b = generated.shape[0]
    x = (generated.reshape(b, G, PS, G, PS)
         .transpose(2, 4, 0, 1, 3)
         .reshape(NPIX, b * G * G))

    tgts = (tgt_s0.reshape(b, NB), tgt_s1.reshape(b * 4, NB),
            tgt_s2.reshape(b * 16, NB), tgt_s3.reshape(b * 64, NB))

    out = pl.pallas_call(
        _fused_body,
        grid=(NPAT // 128, 4),
        in_specs=[
            *(pl.BlockSpec((t.shape[0], NB), lambda j, k: (0, 0))
              for t in tgts),
            pl.BlockSpec((NPIX, 128), lambda j, k: (0, j)),
            pl.BlockSpec((ROWS_PAD, NPAT), lambda j, k: (0, 0)),
        ],
        out_specs=pl.BlockSpec((1, 1), lambda j, k: (0, 0)),
        out_shape=jax.ShapeDtypeStruct((1, 1), jnp.float32),
        scratch_shapes=[pltpu.VMEM((NB, NPAT), jnp.float32),
                        pltpu.VMEM((ROWS_PAD, NB), jnp.float32)],
        compiler_params=pltpu.CompilerParams(
            dimension_semantics=("arbitrary", "arbitrary")),
    )(*tgts, x, jnp.asarray(_A_POOL))
    return out[0, 0]
